# baseline (speedup 1.0000x reference)
import jax
import jax.numpy as jnp
from jax import lax
from jax.experimental import pallas as pl
from jax.experimental.pallas import tpu as pltpu

_B = 4
_S = 2048
_D = 4096
_R = 16


def _tc_body(ids_ref, hidden_hbm, out_ref, sem):
    rr = lax.broadcasted_iota(jnp.int32, (_B * _R, 128), 0)
    cc = lax.broadcasted_iota(jnp.int32, (_B * _R, 128), 1)
    pos = (rr & (_R - 1)) * 128 + cc
    key = ids_ref[...] * _S + ((_S - 1) - pos)
    copies = []
    for b in range(_B):
        best = jnp.max(key[b * _R : (b + 1) * _R, :])
        idx = (_S - 1) - (best & (_S - 1))
        copy = pltpu.make_async_copy(
            hidden_hbm.at[pl.ds(b * _S + idx, 1), :],
            out_ref.at[pl.ds(b, 1), :],
            sem,
        )
        copy.start()
        copies.append(copy)
    for copy in copies:
        copy.wait()


@jax.jit
def kernel(last_hidden_state, input_ids):
    hidden2d = last_hidden_state.reshape(_B * _S, _D)
    ids = input_ids.reshape(_B * _R, 128)
    return pl.pallas_call(
        _tc_body,
        out_shape=jax.ShapeDtypeStruct((_B, _D), jnp.float32),
        in_specs=[
            pl.BlockSpec(memory_space=pltpu.VMEM),
            pl.BlockSpec(memory_space=pltpu.MemorySpace.HBM),
        ],
        out_specs=pl.BlockSpec(memory_space=pltpu.VMEM),
        scratch_shapes=[pltpu.SemaphoreType.DMA],
    )(ids, hidden2d)
